# baseline (speedup 1.0000x reference)
import functools
import math

import numpy as np
import jax
import jax.numpy as jnp
from jax import lax
from jax.experimental import pallas as pl
from jax.experimental.pallas import tpu as pltpu
from jax.experimental.pallas import tpu_sc as plsc

TEXT_MAX_LEN = 2048
D_MODEL = 768
EPS = 1e-05

_NC = 2
_NS = 16
_NW = _NC * _NS


def _position_encoding_np(length, d_model, min_timescale=1.0, max_timescale=10000.0):
    position = np.arange(length, dtype=np.float32)
    num_timescales = d_model // 2
    log_timescale_increment = math.log(float(max_timescale) / float(min_timescale)) / (
        float(num_timescales) - 1.0
    )
    inv_timescales = min_timescale * np.exp(
        np.arange(num_timescales, dtype=np.float32) * -log_timescale_increment
    )
    scaled_time = position[:, None] * inv_timescales[None, :]
    return np.concatenate(
        [np.sin(scaled_time), np.cos(scaled_time)], axis=1
    ).astype(np.float32)


_POS = _position_encoding_np(TEXT_MAX_LEN, D_MODEL)


def _make_sc_gather(bsz, seq, d):
    n_tokens = bsz * seq
    assert n_tokens % _NW == 0
    per_w = n_tokens // _NW
    assert seq % per_w == 0
    nbuf = 4
    chunk = 32
    n_chunks = per_w // chunk

    mesh = plsc.VectorSubcoreMesh(core_axis_name="c", subcore_axis_name="s")

    @functools.partial(
        pl.kernel,
        mesh=mesh,
        out_type=jax.ShapeDtypeStruct((n_tokens, d), jnp.float32),
        scratch_types=[
            pltpu.VMEM((per_w,), jnp.int32),
            pltpu.VMEM((nbuf, chunk, d), jnp.float32),
        ]
        + [pltpu.SemaphoreType.DMA] * (2 * nbuf),
    )
    def gather_kernel(idx_hbm, table_hbm, out_hbm, idx_v, buf_v, *sems):
        gsem = sems[:nbuf]
        osem = sems[nbuf:]
        wid = lax.axis_index("s") * _NC + lax.axis_index("c")
        base = wid * per_w
        row = wid // (seq // per_w)
        col = lax.rem(base, seq)
        pltpu.sync_copy(idx_hbm.at[row, pl.ds(col, per_w)], idx_v)

        gcp = [None] * nbuf
        ocp = [None] * nbuf

        def fire(c):
            b = c % nbuf
            gcp[b] = pltpu.async_copy(
                table_hbm.at[idx_v.at[pl.ds(c * chunk, chunk)]],
                buf_v.at[b],
                gsem[b],
            )

        for c in range(min(nbuf, n_chunks)):
            fire(c)
        for c in range(n_chunks):
            b = c % nbuf
            gcp[b].wait()
            ocp[b] = pltpu.async_copy(
                buf_v.at[b], out_hbm.at[pl.ds(base + c * chunk, chunk)], osem[b]
            )
            nxt = c + nbuf
            if nxt < n_chunks:
                ocp[b].wait()
                fire(nxt)
        for c in range(max(0, n_chunks - nbuf), n_chunks):
            ocp[c % nbuf].wait()

    return gather_kernel


def _tc_addln_compute(x_ref, pos_ref, g_ref, b_ref, o_ref):
    x = x_ref[0] + pos_ref[...]
    mean = jnp.mean(x, axis=-1, keepdims=True)
    xc = x - mean
    var = jnp.mean(xc * xc, axis=-1, keepdims=True)
    o_ref[0] = xc * lax.rsqrt(var + EPS) * g_ref[...] + b_ref[...]


def _tc_addln_body(x_ref, pos_ref, g_ref, b_ref, o_ref):
    _tc_addln_compute(x_ref, pos_ref, g_ref, b_ref, o_ref)


def _tc_addln_alias_body(canvas_ref, x_ref, pos_ref, g_ref, b_ref, o_ref):
    del canvas_ref
    _tc_addln_compute(x_ref, pos_ref, g_ref, b_ref, o_ref)


def _tc_addln_into(part, pos, gamma, beta, canvas, boff, bsz, seq):
    bs, _, d = part.shape
    grid = (bs,)
    data_specs = [
        pl.BlockSpec((1, seq, d), lambda b: (b, 0, 0)),
        pl.BlockSpec((seq, d), lambda b: (0, 0)),
        pl.BlockSpec((1, d), lambda b: (0, 0)),
        pl.BlockSpec((1, d), lambda b: (0, 0)),
    ]
    out_spec = pl.BlockSpec((1, seq, d), lambda b: (b + boff, 0, 0))
    out_shape = jax.ShapeDtypeStruct((bsz, seq, d), jnp.float32)
    if canvas is None:
        return pl.pallas_call(
            _tc_addln_body,
            grid=grid,
            in_specs=data_specs,
            out_specs=out_spec,
            out_shape=out_shape,
        )(part, pos, gamma, beta)
    return pl.pallas_call(
        _tc_addln_alias_body,
        grid=grid,
        in_specs=[pl.BlockSpec(memory_space=pl.ANY)] + data_specs,
        out_specs=out_spec,
        out_shape=out_shape,
        input_output_aliases={0: 0},
    )(canvas, part, pos, gamma, beta)


_NSPLIT = 4


def kernel(inputs, table, ln_gamma, ln_beta):
    bsz, seq = inputs.shape
    _, d = table.shape
    idx = inputs.astype(jnp.int32)
    pos = jnp.asarray(_POS)
    g = ln_gamma.reshape(1, d)
    b = ln_beta.reshape(1, d)
    bs = bsz // _NSPLIT
    gather = _make_sc_gather(bs, seq, d)
    parts = [
        gather(idx[i * bs:(i + 1) * bs], table).reshape(bs, seq, d)
        for i in range(_NSPLIT)
    ]
    out = None
    for i, part in enumerate(parts):
        out = _tc_addln_into(part, pos, g, b, out, i * bs, bsz, seq)
    return out


# speedup vs baseline: 1.1773x; 1.1773x over previous
import functools
import math

import numpy as np
import jax
import jax.numpy as jnp
from jax import lax
from jax.experimental import pallas as pl
from jax.experimental.pallas import tpu as pltpu
from jax.experimental.pallas import tpu_sc as plsc

TEXT_MAX_LEN = 2048
D_MODEL = 768
EPS = 1e-05

_NC = 2
_NS = 16
_NW = _NC * _NS


def _position_encoding_np(length, d_model, min_timescale=1.0, max_timescale=10000.0):
    position = np.arange(length, dtype=np.float32)
    num_timescales = d_model // 2
    log_timescale_increment = math.log(float(max_timescale) / float(min_timescale)) / (
        float(num_timescales) - 1.0
    )
    inv_timescales = min_timescale * np.exp(
        np.arange(num_timescales, dtype=np.float32) * -log_timescale_increment
    )
    scaled_time = position[:, None] * inv_timescales[None, :]
    return np.concatenate(
        [np.sin(scaled_time), np.cos(scaled_time)], axis=1
    ).astype(np.float32)


_POS = _position_encoding_np(TEXT_MAX_LEN, D_MODEL)


def _make_sc_gather(bsz, seq, d):
    n_tokens = bsz * seq
    assert n_tokens % _NW == 0
    per_w = n_tokens // _NW
    assert seq % per_w == 0
    nbuf = 4
    chunk = 32
    n_chunks = per_w // chunk

    mesh = plsc.VectorSubcoreMesh(core_axis_name="c", subcore_axis_name="s")

    @functools.partial(
        pl.kernel,
        mesh=mesh,
        out_type=jax.ShapeDtypeStruct((n_tokens, d), jnp.float32),
        scratch_types=[
            pltpu.VMEM((per_w,), jnp.int32),
            pltpu.VMEM((nbuf, chunk, d), jnp.float32),
        ]
        + [pltpu.SemaphoreType.DMA] * (2 * nbuf),
    )
    def gather_kernel(idx_hbm, table_hbm, out_hbm, idx_v, buf_v, *sems):
        gsem = sems[:nbuf]
        osem = sems[nbuf:]
        wid = lax.axis_index("s") * _NC + lax.axis_index("c")
        base = wid * per_w
        row = wid // (seq // per_w)
        col = lax.rem(base, seq)
        pltpu.sync_copy(idx_hbm.at[row, pl.ds(col, per_w)], idx_v)

        gcp = [None] * nbuf
        ocp = [None] * nbuf

        def fire(c):
            b = c % nbuf
            gcp[b] = pltpu.async_copy(
                table_hbm.at[idx_v.at[pl.ds(c * chunk, chunk)]],
                buf_v.at[b],
                gsem[b],
            )

        for c in range(min(nbuf, n_chunks)):
            fire(c)
        for c in range(n_chunks):
            b = c % nbuf
            gcp[b].wait()
            ocp[b] = pltpu.async_copy(
                buf_v.at[b], out_hbm.at[pl.ds(base + c * chunk, chunk)], osem[b]
            )
            nxt = c + nbuf
            if nxt < n_chunks:
                ocp[b].wait()
                fire(nxt)
        for c in range(max(0, n_chunks - nbuf), n_chunks):
            ocp[c % nbuf].wait()

    return gather_kernel


def _tc_addln_compute(x_ref, pos_ref, g_ref, b_ref, o_ref):
    x = x_ref[0] + pos_ref[...]
    mean = jnp.mean(x, axis=-1, keepdims=True)
    xc = x - mean
    var = jnp.mean(xc * xc, axis=-1, keepdims=True)
    o_ref[0] = xc * lax.rsqrt(var + EPS) * g_ref[...] + b_ref[...]


def _tc_addln_body(x_ref, pos_ref, g_ref, b_ref, o_ref):
    _tc_addln_compute(x_ref, pos_ref, g_ref, b_ref, o_ref)


def _tc_addln_alias_body(canvas_ref, x_ref, pos_ref, g_ref, b_ref, o_ref):
    del canvas_ref
    _tc_addln_compute(x_ref, pos_ref, g_ref, b_ref, o_ref)


def _tc_addln_into(part, pos, gamma, beta, canvas, boff, bsz, seq):
    bs, _, d = part.shape
    grid = (bs,)
    data_specs = [
        pl.BlockSpec((1, seq, d), lambda b: (b, 0, 0)),
        pl.BlockSpec((seq, d), lambda b: (0, 0)),
        pl.BlockSpec((1, d), lambda b: (0, 0)),
        pl.BlockSpec((1, d), lambda b: (0, 0)),
    ]
    out_spec = pl.BlockSpec((1, seq, d), lambda b: (b + boff, 0, 0))
    out_shape = jax.ShapeDtypeStruct((bsz, seq, d), jnp.float32)
    if canvas is None:
        return pl.pallas_call(
            _tc_addln_body,
            grid=grid,
            in_specs=data_specs,
            out_specs=out_spec,
            out_shape=out_shape,
        )(part, pos, gamma, beta)
    return pl.pallas_call(
        _tc_addln_alias_body,
        grid=grid,
        in_specs=[pl.BlockSpec(memory_space=pl.ANY)] + data_specs,
        out_specs=out_spec,
        out_shape=out_shape,
        input_output_aliases={0: 0},
    )(canvas, part, pos, gamma, beta)


_NSPLIT = 1


def kernel(inputs, table, ln_gamma, ln_beta):
    bsz, seq = inputs.shape
    _, d = table.shape
    idx = inputs.astype(jnp.int32)
    pos = jnp.asarray(_POS)
    g = ln_gamma.reshape(1, d)
    b = ln_beta.reshape(1, d)
    bs = bsz // _NSPLIT
    gather = _make_sc_gather(bs, seq, d)
    parts = [
        gather(idx[i * bs:(i + 1) * bs], table).reshape(bs, seq, d)
        for i in range(_NSPLIT)
    ]
    out = None
    for i, part in enumerate(parts):
        out = _tc_addln_into(part, pos, g, b, out, i * bs, bsz, seq)
    return out


# speedup vs baseline: 1.1928x; 1.0132x over previous
import functools
import math

import numpy as np
import jax
import jax.numpy as jnp
from jax import lax
from jax.experimental import pallas as pl
from jax.experimental.pallas import tpu as pltpu
from jax.experimental.pallas import tpu_sc as plsc

TEXT_MAX_LEN = 2048
D_MODEL = 768
EPS = 1e-05

_NC = 2
_NS = 16
_NW = _NC * _NS


def _position_encoding_np(length, d_model, min_timescale=1.0, max_timescale=10000.0):
    position = np.arange(length, dtype=np.float32)
    num_timescales = d_model // 2
    log_timescale_increment = math.log(float(max_timescale) / float(min_timescale)) / (
        float(num_timescales) - 1.0
    )
    inv_timescales = min_timescale * np.exp(
        np.arange(num_timescales, dtype=np.float32) * -log_timescale_increment
    )
    scaled_time = position[:, None] * inv_timescales[None, :]
    return np.concatenate(
        [np.sin(scaled_time), np.cos(scaled_time)], axis=1
    ).astype(np.float32)


_POS = _position_encoding_np(TEXT_MAX_LEN, D_MODEL)


def _make_sc_gather(bsz, seq, d):
    n_tokens = bsz * seq
    assert n_tokens % _NW == 0
    per_w = n_tokens // _NW
    assert seq % per_w == 0
    nbuf = 4
    chunk = 32
    n_chunks = per_w // chunk

    mesh = plsc.VectorSubcoreMesh(core_axis_name="c", subcore_axis_name="s")

    @functools.partial(
        pl.kernel,
        mesh=mesh,
        out_type=jax.ShapeDtypeStruct((n_tokens, d), jnp.float32),
        scratch_types=[
            pltpu.VMEM((per_w,), jnp.int32),
            pltpu.VMEM((nbuf, chunk, d), jnp.float32),
        ]
        + [pltpu.SemaphoreType.DMA] * (2 * nbuf),
    )
    def gather_kernel(idx_hbm, table_hbm, out_hbm, idx_v, buf_v, *sems):
        gsem = sems[:nbuf]
        osem = sems[nbuf:]
        wid = lax.axis_index("s") * _NC + lax.axis_index("c")
        base = wid * per_w
        row = wid // (seq // per_w)
        col = lax.rem(base, seq)
        pltpu.sync_copy(idx_hbm.at[row, pl.ds(col, per_w)], idx_v)

        gcp = [None] * nbuf
        ocp = [None] * nbuf

        def fire(c):
            b = c % nbuf
            gcp[b] = pltpu.async_copy(
                table_hbm.at[idx_v.at[pl.ds(c * chunk, chunk)]],
                buf_v.at[b],
                gsem[b],
            )

        for c in range(min(nbuf, n_chunks)):
            fire(c)
        for c in range(n_chunks):
            b = c % nbuf
            gcp[b].wait()
            ocp[b] = pltpu.async_copy(
                buf_v.at[b], out_hbm.at[pl.ds(base + c * chunk, chunk)], osem[b]
            )
            nxt = c + nbuf
            if nxt < n_chunks:
                ocp[b].wait()
                fire(nxt)
        for c in range(max(0, n_chunks - nbuf), n_chunks):
            ocp[c % nbuf].wait()

    return gather_kernel


def _tc_addln_body(x_ref, pos_ref, g_ref, b_ref, o_ref):
    x = x_ref[0] + pos_ref[...]
    mean = jnp.mean(x, axis=-1, keepdims=True)
    xc = x - mean
    var = jnp.mean(xc * xc, axis=-1, keepdims=True)
    o_ref[0] = xc * lax.rsqrt(var + EPS) * g_ref[...] + b_ref[...]


def _tc_addln(gathered, pos, gamma, beta, bsz, seq):
    _, d = gathered.shape
    return pl.pallas_call(
        _tc_addln_body,
        grid=(bsz,),
        in_specs=[
            pl.BlockSpec((1, seq, d), lambda b: (b, 0, 0)),
            pl.BlockSpec((seq, d), lambda b: (0, 0)),
            pl.BlockSpec((1, d), lambda b: (0, 0)),
            pl.BlockSpec((1, d), lambda b: (0, 0)),
        ],
        out_specs=pl.BlockSpec((1, seq, d), lambda b: (b, 0, 0)),
        out_shape=jax.ShapeDtypeStruct((bsz, seq, d), jnp.float32),
    )(gathered.reshape(bsz, seq, d), pos, gamma, beta)


def kernel(inputs, table, ln_gamma, ln_beta):
    bsz, seq = inputs.shape
    _, d = table.shape
    idx = inputs.astype(jnp.int32)
    gathered = _make_sc_gather(bsz, seq, d)(idx, table)
    pos = jnp.asarray(_POS)
    return _tc_addln(
        gathered, pos, ln_gamma.reshape(1, d), ln_beta.reshape(1, d), bsz, seq
    )


# speedup vs baseline: 1.1971x; 1.0036x over previous
import functools
import math

import numpy as np
import jax
import jax.numpy as jnp
from jax import lax
from jax.experimental import pallas as pl
from jax.experimental.pallas import tpu as pltpu
from jax.experimental.pallas import tpu_sc as plsc

TEXT_MAX_LEN = 2048
D_MODEL = 768
EPS = 1e-05

_NC = 2
_NS = 16
_NW = _NC * _NS


def _position_encoding_np(length, d_model, min_timescale=1.0, max_timescale=10000.0):
    position = np.arange(length, dtype=np.float32)
    num_timescales = d_model // 2
    log_timescale_increment = math.log(float(max_timescale) / float(min_timescale)) / (
        float(num_timescales) - 1.0
    )
    inv_timescales = min_timescale * np.exp(
        np.arange(num_timescales, dtype=np.float32) * -log_timescale_increment
    )
    scaled_time = position[:, None] * inv_timescales[None, :]
    return np.concatenate(
        [np.sin(scaled_time), np.cos(scaled_time)], axis=1
    ).astype(np.float32)


_POS = _position_encoding_np(TEXT_MAX_LEN, D_MODEL).astype(jnp.bfloat16)


def _make_sc_gather(bsz, seq, d):
    n_tokens = bsz * seq
    assert n_tokens % _NW == 0
    per_w = n_tokens // _NW
    assert seq % per_w == 0
    nbuf = 4
    chunk = 32
    n_chunks = per_w // chunk

    mesh = plsc.VectorSubcoreMesh(core_axis_name="c", subcore_axis_name="s")

    @functools.partial(
        pl.kernel,
        mesh=mesh,
        out_type=jax.ShapeDtypeStruct((n_tokens, d), jnp.float32),
        scratch_types=[
            pltpu.VMEM((per_w,), jnp.int32),
            pltpu.VMEM((nbuf, chunk, d), jnp.float32),
        ]
        + [pltpu.SemaphoreType.DMA] * (2 * nbuf),
    )
    def gather_kernel(idx_hbm, table_hbm, out_hbm, idx_v, buf_v, *sems):
        gsem = sems[:nbuf]
        osem = sems[nbuf:]
        wid = lax.axis_index("s") * _NC + lax.axis_index("c")
        base = wid * per_w
        row = wid // (seq // per_w)
        col = lax.rem(base, seq)
        pltpu.sync_copy(idx_hbm.at[row, pl.ds(col, per_w)], idx_v)

        gcp = [None] * nbuf
        ocp = [None] * nbuf

        def fire(c):
            b = c % nbuf
            gcp[b] = pltpu.async_copy(
                table_hbm.at[idx_v.at[pl.ds(c * chunk, chunk)]],
                buf_v.at[b],
                gsem[b],
            )

        for c in range(min(nbuf, n_chunks)):
            fire(c)
        for c in range(n_chunks):
            b = c % nbuf
            gcp[b].wait()
            ocp[b] = pltpu.async_copy(
                buf_v.at[b], out_hbm.at[pl.ds(base + c * chunk, chunk)], osem[b]
            )
            nxt = c + nbuf
            if nxt < n_chunks:
                ocp[b].wait()
                fire(nxt)
        for c in range(max(0, n_chunks - nbuf), n_chunks):
            ocp[c % nbuf].wait()

    return gather_kernel


def _tc_addln_body(x_ref, pos_ref, g_ref, b_ref, o_ref):
    x = x_ref[0] + pos_ref[...].astype(jnp.float32)
    mean = jnp.mean(x, axis=-1, keepdims=True)
    xc = x - mean
    var = jnp.mean(xc * xc, axis=-1, keepdims=True)
    o_ref[0] = xc * lax.rsqrt(var + EPS) * g_ref[...] + b_ref[...]


def _tc_addln(gathered, pos, gamma, beta, bsz, seq):
    _, d = gathered.shape
    return pl.pallas_call(
        _tc_addln_body,
        grid=(bsz,),
        in_specs=[
            pl.BlockSpec((1, seq, d), lambda b: (b, 0, 0)),
            pl.BlockSpec((seq, d), lambda b: (0, 0)),
            pl.BlockSpec((1, d), lambda b: (0, 0)),
            pl.BlockSpec((1, d), lambda b: (0, 0)),
        ],
        out_specs=pl.BlockSpec((1, seq, d), lambda b: (b, 0, 0)),
        out_shape=jax.ShapeDtypeStruct((bsz, seq, d), jnp.float32),
    )(gathered.reshape(bsz, seq, d), pos, gamma, beta)


def kernel(inputs, table, ln_gamma, ln_beta):
    bsz, seq = inputs.shape
    _, d = table.shape
    idx = inputs.astype(jnp.int32)
    gathered = _make_sc_gather(bsz, seq, d)(idx, table)
    pos = jnp.asarray(_POS)
    return _tc_addln(
        gathered, pos, ln_gamma.reshape(1, d), ln_beta.reshape(1, d), bsz, seq
    )
